# baseline (speedup 1.0000x reference)
import functools

import jax
import jax.numpy as jnp
from jax import lax
from jax.experimental import pallas as pl
from jax.experimental.pallas import tpu as pltpu
from jax.experimental.pallas import tpu_sc as plsc

NC = 2
NS = 16
NW = NC * NS
L = 16

IMAX = 2**31 - 1

XROWS_PER_W = 2
XCHUNK = 4096
XNCHUNK = 32768 // XCHUNK
XTOT = XROWS_PER_W * XNCHUNK
YGROUPS = 8
YCHUNK = 512
YNCHUNK = 4096 // YCHUNK
YB = 128
YBPC = YCHUNK // YB
YNBLK = 4096 // YB
ZBLK = 64
ZNBLK = 2048 // ZBLK


def _insert2_desc(a, ia, b, ib, v, iv):
    gt1 = v > a
    gt2 = v > b
    na = jnp.where(gt1, v, a)
    nia = jnp.where(gt1, iv, ia)
    nb = jnp.where(gt1, a, jnp.where(gt2, v, b))
    nib = jnp.where(gt1, ia, jnp.where(gt2, iv, ib))
    return na, nia, nb, nib


def _insert3_desc(t0, i0, t1, i1, t2, i2, v, iv):
    c0 = v > t0
    c1 = v > t1
    c2 = v > t2
    nt2 = jnp.where(c2, jnp.where(c1, t1, v), t2)
    ni2 = jnp.where(c2, jnp.where(c1, i1, iv), i2)
    nt1 = jnp.where(c1, jnp.where(c0, t0, v), t1)
    ni1 = jnp.where(c1, jnp.where(c0, i0, iv), i1)
    nt0 = jnp.where(c0, v, t0)
    ni0 = jnp.where(c0, iv, i0)
    return nt0, ni0, nt1, ni1, nt2, ni2


def _insert4_asc(t0, i0, t1, i1, t2, i2, t3, i3, v, iv):
    c0 = v < t0
    c1 = v < t1
    c2 = v < t2
    c3 = v < t3
    nt3 = jnp.where(c3, jnp.where(c2, t2, v), t3)
    ni3 = jnp.where(c3, jnp.where(c2, i2, iv), i3)
    nt2 = jnp.where(c2, jnp.where(c1, t1, v), t2)
    ni2 = jnp.where(c2, jnp.where(c1, i1, iv), i2)
    nt1 = jnp.where(c1, jnp.where(c0, t0, v), t1)
    ni1 = jnp.where(c1, jnp.where(c0, i0, iv), i1)
    nt0 = jnp.where(c0, v, t0)
    ni0 = jnp.where(c0, iv, i0)
    return nt0, ni0, nt1, ni1, nt2, ni2, nt3, ni3


def _bcast_reduce(v, op, iota):
    for d in (8, 4, 2, 1):
        perm = jnp.bitwise_xor(iota, d)
        v = op(v, v.at[perm].get(mode="promise_in_bounds"))
    return v


def _sc_body(x_hbm, y_hbm, yf_hbm, z_hbm, xv_o, xi_o, yv_o, yi_o, zv_o, zi_o,
             xbuf, ybuf, btab, rbuf, zbuf, zsv, zsi, obf, obi, sems):
    c = lax.axis_index("c")
    s = lax.axis_index("s")
    w = s * NC + c
    iota = lax.iota(jnp.int32, L)
    ninf = jnp.full((L,), -jnp.inf, jnp.float32)
    pinf = jnp.full((L,), jnp.inf, jnp.float32)
    zero_i = jnp.zeros((L,), jnp.int32)

    phase_x = jax.named_scope("phase_x"); phase_x.__enter__()
    def xsrc(t):
        row = w * XROWS_PER_W + t // XNCHUNK
        return x_hbm.at[row, pl.ds((t % XNCHUNK) * XCHUNK, XCHUNK)]

    pltpu.async_copy(xsrc(0), xbuf.at[0], sems.at[0])

    xres_v = jnp.zeros((L,), jnp.float32)
    xres_i = jnp.zeros((L,), jnp.int32)
    for rr in range(XROWS_PER_W):

        def xchunk(ch, carry):
            a, ia, b, ib = carry
            t = rr * XNCHUNK + ch
            cur = t & 1
            pltpu.make_async_copy(xsrc(t), xbuf.at[cur], sems.at[cur]).wait()

            @pl.when(t + 1 < XTOT)
            def _prefetch():
                pltpu.async_copy(xsrc(t + 1), xbuf.at[1 - cur], sems.at[1 - cur])

            base = ch * XCHUNK

            def xstep(i, carry):
                a, ia, b, ib = carry
                v = xbuf[cur, pl.ds(i * L, L)]
                iv = iota + (base + i * L)
                return _insert2_desc(a, ia, b, ib, v, iv)

            return lax.fori_loop(0, XCHUNK // L, xstep, (a, ia, b, ib))

        a, ia, b, ib = lax.fori_loop(
            0, XNCHUNK, xchunk, (ninf, zero_i, ninf, zero_i))
        m1 = _bcast_reduce(a, jnp.maximum, iota)
        im1 = _bcast_reduce(jnp.where(a == m1, ia, IMAX), jnp.minimum, iota)
        cnd = jnp.where(ia == im1, b, a)
        icnd = jnp.where(ia == im1, ib, ia)
        m2 = _bcast_reduce(cnd, jnp.maximum, iota)
        im2 = _bcast_reduce(jnp.where(cnd == m2, icnd, IMAX), jnp.minimum, iota)
        xres_v = jnp.where(iota == 2 * rr, m1, jnp.where(iota == 2 * rr + 1, m2, xres_v))
        xres_i = jnp.where(iota == 2 * rr, im1, jnp.where(iota == 2 * rr + 1, im2, xres_i))
    obf[pl.ds(0, L)] = xres_v
    obi[pl.ds(0, L)] = xres_i
    pltpu.sync_copy(obf.at[pl.ds(0, L)], xv_o.at[pl.ds(w * L, L)])
    pltpu.sync_copy(obi.at[pl.ds(0, L)], xi_o.at[pl.ds(w * L, L)])

    phase_x.__exit__(None, None, None)
    phase_y = jax.named_scope("phase_y"); phase_y.__enter__()
    YTOT = YGROUPS * YNCHUNK

    def ysrc(t):
        g = t // YNCHUNK
        ch = t % YNCHUNK
        r0 = w * (YGROUPS * L) + g * L
        return y_hbm.at[pl.ds(r0, L), pl.ds(ch * YCHUNK, YCHUNK)]

    pltpu.async_copy(ysrc(0), ybuf.at[0, :, pl.ds(0, YCHUNK)], sems.at[0])

    def ygroup(g, _):
        r0 = w * (YGROUPS * L) + g * L

        def ychunk(ch, _):
            t = g * YNCHUNK + ch
            cur = t & 1
            pltpu.make_async_copy(
                ysrc(t), ybuf.at[cur, :, pl.ds(0, YCHUNK)], sems.at[cur]).wait()

            @pl.when(t + 1 < YTOT)
            def _prefetch():
                pltpu.async_copy(
                    ysrc(t + 1), ybuf.at[1 - cur, :, pl.ds(0, YCHUNK)],
                    sems.at[1 - cur])

            curv = jnp.full((L,), cur, jnp.int32)

            def yblock(b, _):
                cb = b * YB
                v0 = plsc.load_gather(ybuf, [curv, iota, jnp.full((L,), cb, jnp.int32)])

                def ymin(j, carry):
                    bm, v = carry
                    jn = jnp.minimum(cb + j + 1, YCHUNK - 1)
                    vnext = plsc.load_gather(
                        ybuf, [curv, iota, jnp.full((L,), jn, jnp.int32)])
                    return jnp.minimum(bm, v), vnext

                bm, _v = lax.fori_loop(0, YB, ymin, (pinf, v0))
                btab[ch * YBPC + b] = bm
                return _

            lax.fori_loop(0, YBPC, yblock, 0)
            return _

        lax.fori_loop(0, YNCHUNK, ychunk, 0)

        def ysel(blk, carry):
            t0, i0, t1, i1, t2, i2, t3, i3 = carry
            bm = btab[blk]
            bi = jnp.full((L,), blk, jnp.int32)
            return _insert4_asc(t0, i0, t1, i1, t2, i2, t3, i3, bm, bi)

        sel = lax.fori_loop(
            0, YNBLK, ysel,
            (pinf, zero_i, pinf, zero_i, pinf, zero_i, pinf, zero_i))
        b0, b1, b2, b3 = sel[1], sel[3], sel[5], sel[7]
        b0, b1 = jnp.minimum(b0, b1), jnp.maximum(b0, b1)
        b2, b3 = jnp.minimum(b2, b3), jnp.maximum(b2, b3)
        b0, b2 = jnp.minimum(b0, b2), jnp.maximum(b0, b2)
        b1, b3 = jnp.minimum(b1, b3), jnp.maximum(b1, b3)
        b1, b2 = jnp.minimum(b1, b2), jnp.maximum(b1, b2)

        rowbase = (r0 + iota) * YNBLK
        chosen = (b0, b1, b2, b3)
        for k in range(4):
            pltpu.async_copy(yf_hbm.at[rowbase + chosen[k]], rbuf.at[k],
                             sems.at[2])
        st = (pinf, zero_i, pinf, zero_i, pinf, zero_i, pinf, zero_i)
        for k in range(4):
            pltpu.make_async_copy(
                yf_hbm.at[rowbase + chosen[k]], rbuf.at[k], sems.at[2]).wait()
            kv = jnp.full((L,), k, jnp.int32)
            cbase = chosen[k] * YB

            def yfine(j, carry):
                t0, i0, t1, i1, t2, i2, t3, i3 = carry
                v = plsc.load_gather(rbuf, [kv, iota, jnp.full((L,), j, jnp.int32)])
                iv = cbase + j
                return _insert4_asc(t0, i0, t1, i1, t2, i2, t3, i3, v, iv)

            st = lax.fori_loop(0, YB, yfine, st)
        t0, i0, t1, i1, t2, i2, t3, i3 = st
        for k, (tk, ik) in enumerate(((t0, i0), (t1, i1), (t2, i2), (t3, i3))):
            plsc.store_scatter(obf, [iota * 4 + k], tk)
            plsc.store_scatter(obi, [iota * 4 + k], ik)
        pltpu.sync_copy(obf, yv_o.at[pl.ds(r0 * 4, 4 * L)])
        pltpu.sync_copy(obi, yi_o.at[pl.ds(r0 * 4, 4 * L)])
        return _

    lax.fori_loop(0, YGROUPS, ygroup, 0)

    phase_y.__exit__(None, None, None)
    phase_z = jax.named_scope("phase_z"); phase_z.__enter__()
    def zinit(i, _):
        zsv[pl.ds(i * L, L)] = ninf
        zsi[pl.ds(i * L, L)] = zero_i
        return _

    lax.fori_loop(0, 3 * 128 // L, zinit, 0)

    def zsrc(t):
        return z_hbm.at[pl.ds(t * ZBLK, ZBLK), pl.ds(w * 128, 128)]

    pltpu.async_copy(zsrc(0), zbuf.at[0], sems.at[0])

    def zblock(blk, _):
        cur = blk & 1
        pltpu.make_async_copy(zsrc(blk), zbuf.at[cur], sems.at[cur]).wait()

        @pl.when(blk + 1 < ZNBLK)
        def _prefetch():
            pltpu.async_copy(zsrc(blk + 1), zbuf.at[1 - cur], sems.at[1 - cur])

        rbase = blk * ZBLK

        def zgroup(g, _):
            t0 = zsv[pl.ds(0 * 128 + g * L, L)]
            t1 = zsv[pl.ds(1 * 128 + g * L, L)]
            t2 = zsv[pl.ds(2 * 128 + g * L, L)]
            i0 = zsi[pl.ds(0 * 128 + g * L, L)]
            i1 = zsi[pl.ds(1 * 128 + g * L, L)]
            i2 = zsi[pl.ds(2 * 128 + g * L, L)]

            v0 = zbuf[cur, 0, pl.ds(g * L, L)]

            def zstep(r, carry):
                t0, i0, t1, i1, t2, i2, v = carry
                rn = jnp.minimum(r + 1, ZBLK - 1)
                vnext = zbuf[cur, rn, pl.ds(g * L, L)]
                iv = jnp.full((L,), rbase + r, jnp.int32)
                out = _insert3_desc(t0, i0, t1, i1, t2, i2, v, iv)
                return out + (vnext,)

            t0, i0, t1, i1, t2, i2, _vlast = lax.fori_loop(
                0, ZBLK, zstep, (t0, i0, t1, i1, t2, i2, v0))
            zsv[pl.ds(0 * 128 + g * L, L)] = t0
            zsv[pl.ds(1 * 128 + g * L, L)] = t1
            zsv[pl.ds(2 * 128 + g * L, L)] = t2
            zsi[pl.ds(0 * 128 + g * L, L)] = i0
            zsi[pl.ds(1 * 128 + g * L, L)] = i1
            zsi[pl.ds(2 * 128 + g * L, L)] = i2
            return _

        lax.fori_loop(0, 8, zgroup, 0)
        return _

    lax.fori_loop(0, ZNBLK, zblock, 0)
    pltpu.sync_copy(zsv, zv_o.at[pl.ds(w * 384, 384)])
    pltpu.sync_copy(zsi, zi_o.at[pl.ds(w * 384, 384)])
    phase_z.__exit__(None, None, None)


@jax.jit
def _sc_topk(x, y2d, yflat, z):
    mesh = plsc.VectorSubcoreMesh(core_axis_name="c", subcore_axis_name="s")
    f = pl.kernel(
        _sc_body,
        mesh=mesh,
        out_type=[
            jax.ShapeDtypeStruct((NW * L,), jnp.float32),
            jax.ShapeDtypeStruct((NW * L,), jnp.int32),
            jax.ShapeDtypeStruct((4096 * 4,), jnp.float32),
            jax.ShapeDtypeStruct((4096 * 4,), jnp.int32),
            jax.ShapeDtypeStruct((NW * 384,), jnp.float32),
            jax.ShapeDtypeStruct((NW * 384,), jnp.int32),
        ],
        scratch_types=[
            pltpu.VMEM((2, XCHUNK), jnp.float32),
            pltpu.VMEM((2, L, YCHUNK + 1), jnp.float32),
            pltpu.VMEM((YNBLK, L), jnp.float32),
            pltpu.VMEM((4, L, YB), jnp.float32),
            pltpu.VMEM((2, ZBLK, 128), jnp.float32),
            pltpu.VMEM((384,), jnp.float32),
            pltpu.VMEM((384,), jnp.int32),
            pltpu.VMEM((4 * L,), jnp.float32),
            pltpu.VMEM((4 * L,), jnp.int32),
            pltpu.SemaphoreType.DMA((3,)),
        ],
        compiler_params=pltpu.CompilerParams(needs_layout_passes=False),
    )
    return f(x, y2d, yflat, z)


def kernel(x, y, z):
    y2d = y.reshape(4096, 4096)
    yflat = y.reshape(4096 * YNBLK, YB)
    xv, xi, yv, yi, zv, zi = _sc_topk(x, y2d, yflat, z)
    x_values = xv.reshape(NW, L)[:, :4].reshape(64, 2)
    x_indices = xi.reshape(NW, L)[:, :4].reshape(64, 2).astype(jnp.int64)
    y_values = yv.reshape(32, 16, 8, 4)
    y_indices = yi.reshape(32, 16, 8, 4).astype(jnp.int64)
    z_values = zv.reshape(NW, 3, 128).transpose(1, 0, 2).reshape(3, 4096)
    z_indices = zi.reshape(NW, 3, 128).transpose(1, 0, 2).reshape(3, 4096).astype(jnp.int64)
    return (x_values, x_indices, y_values, y_indices, z_values, z_indices)


# speedup vs baseline: 1.1223x; 1.1223x over previous
import functools

import jax
import jax.numpy as jnp
from jax import lax
from jax.experimental import pallas as pl
from jax.experimental.pallas import tpu as pltpu
from jax.experimental.pallas import tpu_sc as plsc

NC = 2
NS = 16
NW = NC * NS
L = 16

IMAX = 2**31 - 1

XROWS_PER_W = 2
XCHUNK = 4096
XNCHUNK = 32768 // XCHUNK
XTOT = XROWS_PER_W * XNCHUNK
YGROUPS = 8
YCHUNK = 512
YNCHUNK = 4096 // YCHUNK
YB = 128
YBPC = YCHUNK // YB
YNBLK = 4096 // YB
ZBLK = 64
ZNBLK = 2048 // ZBLK


def _insert2_desc(a, ia, b, ib, v, iv):
    gt1 = v > a
    gt2 = v > b
    na = jnp.where(gt1, v, a)
    nia = jnp.where(gt1, iv, ia)
    nb = jnp.where(gt1, a, jnp.where(gt2, v, b))
    nib = jnp.where(gt1, ia, jnp.where(gt2, iv, ib))
    return na, nia, nb, nib


def _insert3_desc(t0, i0, t1, i1, t2, i2, v, iv):
    c0 = v > t0
    c1 = v > t1
    c2 = v > t2
    nt2 = jnp.where(c2, jnp.where(c1, t1, v), t2)
    ni2 = jnp.where(c2, jnp.where(c1, i1, iv), i2)
    nt1 = jnp.where(c1, jnp.where(c0, t0, v), t1)
    ni1 = jnp.where(c1, jnp.where(c0, i0, iv), i1)
    nt0 = jnp.where(c0, v, t0)
    ni0 = jnp.where(c0, iv, i0)
    return nt0, ni0, nt1, ni1, nt2, ni2


def _insert4_asc(t0, i0, t1, i1, t2, i2, t3, i3, v, iv):
    c0 = v < t0
    c1 = v < t1
    c2 = v < t2
    c3 = v < t3
    nt3 = jnp.where(c3, jnp.where(c2, t2, v), t3)
    ni3 = jnp.where(c3, jnp.where(c2, i2, iv), i3)
    nt2 = jnp.where(c2, jnp.where(c1, t1, v), t2)
    ni2 = jnp.where(c2, jnp.where(c1, i1, iv), i2)
    nt1 = jnp.where(c1, jnp.where(c0, t0, v), t1)
    ni1 = jnp.where(c1, jnp.where(c0, i0, iv), i1)
    nt0 = jnp.where(c0, v, t0)
    ni0 = jnp.where(c0, iv, i0)
    return nt0, ni0, nt1, ni1, nt2, ni2, nt3, ni3


def _bcast_reduce(v, op, iota):
    for d in (8, 4, 2, 1):
        perm = jnp.bitwise_xor(iota, d)
        v = op(v, v.at[perm].get(mode="promise_in_bounds"))
    return v


def _sc_body(x_hbm, y_hbm, z_hbm, xv_o, xi_o, yv_o, yi_o, zv_o, zi_o,
             xbuf, ybuf, btab, zbuf, zsv, zsi, obf, obi, sems):
    c = lax.axis_index("c")
    s = lax.axis_index("s")
    w = s * NC + c
    iota = lax.iota(jnp.int32, L)
    ninf = jnp.full((L,), -jnp.inf, jnp.float32)
    pinf = jnp.full((L,), jnp.inf, jnp.float32)
    zero_i = jnp.zeros((L,), jnp.int32)

    phase_x = jax.named_scope("phase_x"); phase_x.__enter__()
    def xsrc(t):
        row = w * XROWS_PER_W + t // XNCHUNK
        return x_hbm.at[row, pl.ds((t % XNCHUNK) * XCHUNK, XCHUNK)]

    pltpu.async_copy(xsrc(0), xbuf.at[0], sems.at[0])

    xres_v = jnp.zeros((L,), jnp.float32)
    xres_i = jnp.zeros((L,), jnp.int32)
    for rr in range(XROWS_PER_W):

        def xchunk(ch, carry):
            a, ia, b, ib = carry
            t = rr * XNCHUNK + ch
            cur = t & 1
            pltpu.make_async_copy(xsrc(t), xbuf.at[cur], sems.at[cur]).wait()

            @pl.when(t + 1 < XTOT)
            def _prefetch():
                pltpu.async_copy(xsrc(t + 1), xbuf.at[1 - cur], sems.at[1 - cur])

            base = ch * XCHUNK

            def xstep(i, carry):
                a, ia, b, ib = carry
                v = xbuf[cur, pl.ds(i * L, L)]
                iv = iota + (base + i * L)
                return _insert2_desc(a, ia, b, ib, v, iv)

            return lax.fori_loop(0, XCHUNK // L, xstep, (a, ia, b, ib))

        a, ia, b, ib = lax.fori_loop(
            0, XNCHUNK, xchunk, (ninf, zero_i, ninf, zero_i))
        m1 = _bcast_reduce(a, jnp.maximum, iota)
        im1 = _bcast_reduce(jnp.where(a == m1, ia, IMAX), jnp.minimum, iota)
        cnd = jnp.where(ia == im1, b, a)
        icnd = jnp.where(ia == im1, ib, ia)
        m2 = _bcast_reduce(cnd, jnp.maximum, iota)
        im2 = _bcast_reduce(jnp.where(cnd == m2, icnd, IMAX), jnp.minimum, iota)
        xres_v = jnp.where(iota == 2 * rr, m1, jnp.where(iota == 2 * rr + 1, m2, xres_v))
        xres_i = jnp.where(iota == 2 * rr, im1, jnp.where(iota == 2 * rr + 1, im2, xres_i))
    obf[pl.ds(0, L)] = xres_v
    obi[pl.ds(0, L)] = xres_i
    pltpu.sync_copy(obf.at[pl.ds(0, L)], xv_o.at[pl.ds(w * L, L)])
    pltpu.sync_copy(obi.at[pl.ds(0, L)], xi_o.at[pl.ds(w * L, L)])

    phase_x.__exit__(None, None, None)
    phase_y = jax.named_scope("phase_y"); phase_y.__enter__()

    def ygroup(g, _):
        r0 = w * (YGROUPS * L) + g * L

        for ch in range(YNCHUNK):
            pltpu.async_copy(
                y_hbm.at[pl.ds(r0, L), pl.ds(ch * YCHUNK, YCHUNK)],
                ybuf.at[:, pl.ds(ch * YCHUNK, YCHUNK)], sems.at[0])

        def ychunk(ch, _):
            pltpu.make_async_copy(
                y_hbm.at[pl.ds(r0, L), pl.ds(ch * YCHUNK, YCHUNK)],
                ybuf.at[:, pl.ds(ch * YCHUNK, YCHUNK)], sems.at[0]).wait()

            def yblock(b, _):
                cb = ch * YCHUNK + b * YB
                v0 = plsc.load_gather(ybuf, [iota, jnp.full((L,), cb, jnp.int32)])

                def ymin(j, carry):
                    bm, v = carry
                    vnext = plsc.load_gather(
                        ybuf, [iota, jnp.full((L,), cb + jnp.minimum(j + 1, YB - 1), jnp.int32)])
                    return jnp.minimum(bm, v), vnext

                bm, _v = lax.fori_loop(0, YB, ymin, (pinf, v0))
                btab[ch * YBPC + b] = bm
                return _

            lax.fori_loop(0, YBPC, yblock, 0)
            return _

        lax.fori_loop(0, YNCHUNK, ychunk, 0)

        def ysel(blk, carry):
            t0, i0, t1, i1, t2, i2, t3, i3 = carry
            bm = btab[blk]
            bi = jnp.full((L,), blk, jnp.int32)
            return _insert4_asc(t0, i0, t1, i1, t2, i2, t3, i3, bm, bi)

        sel = lax.fori_loop(
            0, YNBLK, ysel,
            (pinf, zero_i, pinf, zero_i, pinf, zero_i, pinf, zero_i))
        b0, b1, b2, b3 = sel[1], sel[3], sel[5], sel[7]
        b0, b1 = jnp.minimum(b0, b1), jnp.maximum(b0, b1)
        b2, b3 = jnp.minimum(b2, b3), jnp.maximum(b2, b3)
        b0, b2 = jnp.minimum(b0, b2), jnp.maximum(b0, b2)
        b1, b3 = jnp.minimum(b1, b3), jnp.maximum(b1, b3)
        b1, b2 = jnp.minimum(b1, b2), jnp.maximum(b1, b2)

        st = (pinf, zero_i, pinf, zero_i, pinf, zero_i, pinf, zero_i)
        for bk in (b0, b1, b2, b3):
            cbase = bk * YB

            def yfine(j, carry, cbase=cbase):
                t0, i0, t1, i1, t2, i2, t3, i3 = carry
                iv = cbase + j
                v = plsc.load_gather(ybuf, [iota, iv])
                return _insert4_asc(t0, i0, t1, i1, t2, i2, t3, i3, v, iv)

            st = lax.fori_loop(0, YB, yfine, st)
        t0, i0, t1, i1, t2, i2, t3, i3 = st
        for k, (tk, ik) in enumerate(((t0, i0), (t1, i1), (t2, i2), (t3, i3))):
            plsc.store_scatter(obf, [iota * 4 + k], tk)
            plsc.store_scatter(obi, [iota * 4 + k], ik)
        pltpu.sync_copy(obf, yv_o.at[pl.ds(r0 * 4, 4 * L)])
        pltpu.sync_copy(obi, yi_o.at[pl.ds(r0 * 4, 4 * L)])
        return _

    lax.fori_loop(0, YGROUPS, ygroup, 0)

    phase_y.__exit__(None, None, None)
    phase_z = jax.named_scope("phase_z"); phase_z.__enter__()
    def zinit(i, _):
        zsv[pl.ds(i * L, L)] = ninf
        zsi[pl.ds(i * L, L)] = zero_i
        return _

    lax.fori_loop(0, 3 * 128 // L, zinit, 0)

    def zsrc(t):
        return z_hbm.at[pl.ds(t * ZBLK, ZBLK), pl.ds(w * 128, 128)]

    pltpu.async_copy(zsrc(0), zbuf.at[0], sems.at[0])

    def zblock(blk, _):
        cur = blk & 1
        pltpu.make_async_copy(zsrc(blk), zbuf.at[cur], sems.at[cur]).wait()

        @pl.when(blk + 1 < ZNBLK)
        def _prefetch():
            pltpu.async_copy(zsrc(blk + 1), zbuf.at[1 - cur], sems.at[1 - cur])

        rbase = blk * ZBLK

        def zgroup(g, _):
            t0 = zsv[pl.ds(0 * 128 + g * L, L)]
            t1 = zsv[pl.ds(1 * 128 + g * L, L)]
            t2 = zsv[pl.ds(2 * 128 + g * L, L)]
            i0 = zsi[pl.ds(0 * 128 + g * L, L)]
            i1 = zsi[pl.ds(1 * 128 + g * L, L)]
            i2 = zsi[pl.ds(2 * 128 + g * L, L)]

            v0 = zbuf[cur, 0, pl.ds(g * L, L)]

            def zstep(r, carry):
                t0, i0, t1, i1, t2, i2, v = carry
                rn = jnp.minimum(r + 1, ZBLK - 1)
                vnext = zbuf[cur, rn, pl.ds(g * L, L)]
                iv = jnp.full((L,), rbase + r, jnp.int32)
                out = _insert3_desc(t0, i0, t1, i1, t2, i2, v, iv)
                return out + (vnext,)

            t0, i0, t1, i1, t2, i2, _vlast = lax.fori_loop(
                0, ZBLK, zstep, (t0, i0, t1, i1, t2, i2, v0))
            zsv[pl.ds(0 * 128 + g * L, L)] = t0
            zsv[pl.ds(1 * 128 + g * L, L)] = t1
            zsv[pl.ds(2 * 128 + g * L, L)] = t2
            zsi[pl.ds(0 * 128 + g * L, L)] = i0
            zsi[pl.ds(1 * 128 + g * L, L)] = i1
            zsi[pl.ds(2 * 128 + g * L, L)] = i2
            return _

        lax.fori_loop(0, 8, zgroup, 0)
        return _

    lax.fori_loop(0, ZNBLK, zblock, 0)
    pltpu.sync_copy(zsv, zv_o.at[pl.ds(w * 384, 384)])
    pltpu.sync_copy(zsi, zi_o.at[pl.ds(w * 384, 384)])
    phase_z.__exit__(None, None, None)


@jax.jit
def _sc_topk(x, y2d, z):
    mesh = plsc.VectorSubcoreMesh(core_axis_name="c", subcore_axis_name="s")
    f = pl.kernel(
        _sc_body,
        mesh=mesh,
        out_type=[
            jax.ShapeDtypeStruct((NW * L,), jnp.float32),
            jax.ShapeDtypeStruct((NW * L,), jnp.int32),
            jax.ShapeDtypeStruct((4096 * 4,), jnp.float32),
            jax.ShapeDtypeStruct((4096 * 4,), jnp.int32),
            jax.ShapeDtypeStruct((NW * 384,), jnp.float32),
            jax.ShapeDtypeStruct((NW * 384,), jnp.int32),
        ],
        scratch_types=[
            pltpu.VMEM((2, XCHUNK), jnp.float32),
            pltpu.VMEM((L, 4096), jnp.float32),
            pltpu.VMEM((YNBLK, L), jnp.float32),
            pltpu.VMEM((2, ZBLK, 128), jnp.float32),
            pltpu.VMEM((384,), jnp.float32),
            pltpu.VMEM((384,), jnp.int32),
            pltpu.VMEM((4 * L,), jnp.float32),
            pltpu.VMEM((4 * L,), jnp.int32),
            pltpu.SemaphoreType.DMA((3,)),
        ],
        compiler_params=pltpu.CompilerParams(needs_layout_passes=False),
    )
    return f(x, y2d, z)


def kernel(x, y, z):
    y2d = y.reshape(4096, 4096)
    xv, xi, yv, yi, zv, zi = _sc_topk(x, y2d, z)
    x_values = xv.reshape(NW, L)[:, :4].reshape(64, 2)
    x_indices = xi.reshape(NW, L)[:, :4].reshape(64, 2).astype(jnp.int64)
    y_values = yv.reshape(32, 16, 8, 4)
    y_indices = yi.reshape(32, 16, 8, 4).astype(jnp.int64)
    z_values = zv.reshape(NW, 3, 128).transpose(1, 0, 2).reshape(3, 4096)
    z_indices = zi.reshape(NW, 3, 128).transpose(1, 0, 2).reshape(3, 4096).astype(jnp.int64)
    return (x_values, x_indices, y_values, y_indices, z_values, z_indices)


# speedup vs baseline: 2.4608x; 2.1925x over previous
import functools

import jax
import jax.numpy as jnp
from jax import lax
from jax.experimental import pallas as pl
from jax.experimental.pallas import tpu as pltpu
from jax.experimental.pallas import tpu_sc as plsc

NC = 2
NS = 16
NW = NC * NS
L = 16

IMAX = 2**31 - 1

XROWS_PER_W = 2
XCHUNK = 4096
XNCHUNK = 32768 // XCHUNK
XTOT = XROWS_PER_W * XNCHUNK
YGROUPS = 8
YCHUNK = 512
YNCHUNK = 4096 // YCHUNK
YB = 128
YBPC = YCHUNK // YB
YNBLK = 4096 // YB
ZBLK = 64
ZNBLK = 2048 // ZBLK


def _insert2_desc(a, ia, b, ib, v, iv):
    gt1 = v > a
    gt2 = v > b
    na = jnp.where(gt1, v, a)
    nia = jnp.where(gt1, iv, ia)
    nb = jnp.where(gt1, a, jnp.where(gt2, v, b))
    nib = jnp.where(gt1, ia, jnp.where(gt2, iv, ib))
    return na, nia, nb, nib


def _insert3_desc(t0, i0, t1, i1, t2, i2, v, iv):
    c0 = v > t0
    c1 = v > t1
    c2 = v > t2
    nt2 = jnp.where(c2, jnp.where(c1, t1, v), t2)
    ni2 = jnp.where(c2, jnp.where(c1, i1, iv), i2)
    nt1 = jnp.where(c1, jnp.where(c0, t0, v), t1)
    ni1 = jnp.where(c1, jnp.where(c0, i0, iv), i1)
    nt0 = jnp.where(c0, v, t0)
    ni0 = jnp.where(c0, iv, i0)
    return nt0, ni0, nt1, ni1, nt2, ni2


def _insert4_asc(t0, i0, t1, i1, t2, i2, t3, i3, v, iv):
    c0 = v < t0
    c1 = v < t1
    c2 = v < t2
    c3 = v < t3
    nt3 = jnp.where(c3, jnp.where(c2, t2, v), t3)
    ni3 = jnp.where(c3, jnp.where(c2, i2, iv), i3)
    nt2 = jnp.where(c2, jnp.where(c1, t1, v), t2)
    ni2 = jnp.where(c2, jnp.where(c1, i1, iv), i2)
    nt1 = jnp.where(c1, jnp.where(c0, t0, v), t1)
    ni1 = jnp.where(c1, jnp.where(c0, i0, iv), i1)
    nt0 = jnp.where(c0, v, t0)
    ni0 = jnp.where(c0, iv, i0)
    return nt0, ni0, nt1, ni1, nt2, ni2, nt3, ni3


def _bcast_reduce(v, op, iota):
    for d in (8, 4, 2, 1):
        perm = jnp.bitwise_xor(iota, d)
        v = op(v, v.at[perm].get(mode="promise_in_bounds"))
    return v


def _sc_body(x_hbm, y_hbm, z_hbm, xv_o, xi_o, yv_o, yi_o, zv_o, zi_o,
             xbuf, ybuf, btab, zbuf, zsv, zsi, obf, obi, sems):
    c = lax.axis_index("c")
    s = lax.axis_index("s")
    w = s * NC + c
    iota = lax.iota(jnp.int32, L)
    ninf = jnp.full((L,), -jnp.inf, jnp.float32)
    pinf = jnp.full((L,), jnp.inf, jnp.float32)
    zero_i = jnp.zeros((L,), jnp.int32)

    phase_x = jax.named_scope("phase_x"); phase_x.__enter__()
    def xsrc(t):
        row = w * XROWS_PER_W + t // XNCHUNK
        return x_hbm.at[row, pl.ds((t % XNCHUNK) * XCHUNK, XCHUNK)]

    pltpu.async_copy(xsrc(0), xbuf.at[0], sems.at[0])

    xres_v = jnp.zeros((L,), jnp.float32)
    xres_i = jnp.zeros((L,), jnp.int32)
    for rr in range(XROWS_PER_W):

        def xchunk(ch, carry):
            a, ia, b, ib = carry
            t = rr * XNCHUNK + ch
            cur = t & 1
            pltpu.make_async_copy(xsrc(t), xbuf.at[cur], sems.at[cur]).wait()

            @pl.when(t + 1 < XTOT)
            def _prefetch():
                pltpu.async_copy(xsrc(t + 1), xbuf.at[1 - cur], sems.at[1 - cur])

            base = ch * XCHUNK

            def xstep(i, carry):
                a, ia, b, ib = carry
                v = xbuf[cur, pl.ds(i * L, L)]
                iv = iota + (base + i * L)
                return _insert2_desc(a, ia, b, ib, v, iv)

            return lax.fori_loop(0, XCHUNK // L, xstep, (a, ia, b, ib))

        a, ia, b, ib = lax.fori_loop(
            0, XNCHUNK, xchunk, (ninf, zero_i, ninf, zero_i))
        m1 = _bcast_reduce(a, jnp.maximum, iota)
        im1 = _bcast_reduce(jnp.where(a == m1, ia, IMAX), jnp.minimum, iota)
        cnd = jnp.where(ia == im1, b, a)
        icnd = jnp.where(ia == im1, ib, ia)
        m2 = _bcast_reduce(cnd, jnp.maximum, iota)
        im2 = _bcast_reduce(jnp.where(cnd == m2, icnd, IMAX), jnp.minimum, iota)
        xres_v = jnp.where(iota == 2 * rr, m1, jnp.where(iota == 2 * rr + 1, m2, xres_v))
        xres_i = jnp.where(iota == 2 * rr, im1, jnp.where(iota == 2 * rr + 1, im2, xres_i))
    obf[pl.ds(0, L)] = xres_v
    obi[pl.ds(0, L)] = xres_i
    pltpu.sync_copy(obf.at[pl.ds(0, L)], xv_o.at[pl.ds(w * L, L)])
    pltpu.sync_copy(obi.at[pl.ds(0, L)], xi_o.at[pl.ds(w * L, L)])

    phase_x.__exit__(None, None, None)
    phase_y = jax.named_scope("phase_y"); phase_y.__enter__()

    def ygroup(g, _):
        r0 = w * (YGROUPS * L) + g * L

        for ch in range(YNCHUNK):
            pltpu.async_copy(
                y_hbm.at[pl.ds(r0, L), pl.ds(ch * YCHUNK, YCHUNK)],
                ybuf.at[:, pl.ds(ch * YCHUNK, YCHUNK)], sems.at[0])

        def ychunk(ch, _):
            pltpu.make_async_copy(
                y_hbm.at[pl.ds(r0, L), pl.ds(ch * YCHUNK, YCHUNK)],
                ybuf.at[:, pl.ds(ch * YCHUNK, YCHUNK)], sems.at[0]).wait()

            def yrow(rr, _):
                def yblock(b, _):
                    cb = ch * YCHUNK + b * YB
                    m0 = jnp.minimum(ybuf[rr, pl.ds(cb, L)],
                                     ybuf[rr, pl.ds(cb + L, L)])
                    m1 = jnp.minimum(ybuf[rr, pl.ds(cb + 2 * L, L)],
                                     ybuf[rr, pl.ds(cb + 3 * L, L)])
                    m2 = jnp.minimum(ybuf[rr, pl.ds(cb + 4 * L, L)],
                                     ybuf[rr, pl.ds(cb + 5 * L, L)])
                    m3 = jnp.minimum(ybuf[rr, pl.ds(cb + 6 * L, L)],
                                     ybuf[rr, pl.ds(cb + 7 * L, L)])
                    bm = _bcast_reduce(
                        jnp.minimum(jnp.minimum(m0, m1), jnp.minimum(m2, m3)),
                        jnp.minimum, iota)
                    plsc.store_scatter(
                        btab, [jnp.full((L,), ch * YBPC + b, jnp.int32), iota],
                        bm, mask=iota == rr)
                    return _

                lax.fori_loop(0, YBPC, yblock, 0)
                return _

            lax.fori_loop(0, L, yrow, 0)
            return _

        lax.fori_loop(0, YNCHUNK, ychunk, 0)

        def ysel(blk, carry):
            t0, i0, t1, i1, t2, i2, t3, i3 = carry
            bm = btab[blk]
            bi = jnp.full((L,), blk, jnp.int32)
            return _insert4_asc(t0, i0, t1, i1, t2, i2, t3, i3, bm, bi)

        sel = lax.fori_loop(
            0, YNBLK, ysel,
            (pinf, zero_i, pinf, zero_i, pinf, zero_i, pinf, zero_i))
        b0, b1, b2, b3 = sel[1], sel[3], sel[5], sel[7]
        b0, b1 = jnp.minimum(b0, b1), jnp.maximum(b0, b1)
        b2, b3 = jnp.minimum(b2, b3), jnp.maximum(b2, b3)
        b0, b2 = jnp.minimum(b0, b2), jnp.maximum(b0, b2)
        b1, b3 = jnp.minimum(b1, b3), jnp.maximum(b1, b3)
        b1, b2 = jnp.minimum(b1, b2), jnp.maximum(b1, b2)

        st = (pinf, zero_i, pinf, zero_i, pinf, zero_i, pinf, zero_i)
        for bk in (b0, b1, b2, b3):
            cbase = bk * YB

            def yfine(j, carry, cbase=cbase):
                t0, i0, t1, i1, t2, i2, t3, i3 = carry
                iv = cbase + j
                v = plsc.load_gather(ybuf, [iota, iv])
                return _insert4_asc(t0, i0, t1, i1, t2, i2, t3, i3, v, iv)

            st = lax.fori_loop(0, YB, yfine, st)
        t0, i0, t1, i1, t2, i2, t3, i3 = st
        for k, (tk, ik) in enumerate(((t0, i0), (t1, i1), (t2, i2), (t3, i3))):
            plsc.store_scatter(obf, [iota * 4 + k], tk)
            plsc.store_scatter(obi, [iota * 4 + k], ik)
        pltpu.sync_copy(obf, yv_o.at[pl.ds(r0 * 4, 4 * L)])
        pltpu.sync_copy(obi, yi_o.at[pl.ds(r0 * 4, 4 * L)])
        return _

    lax.fori_loop(0, YGROUPS, ygroup, 0)

    phase_y.__exit__(None, None, None)
    phase_z = jax.named_scope("phase_z"); phase_z.__enter__()
    def zinit(i, _):
        zsv[pl.ds(i * L, L)] = ninf
        zsi[pl.ds(i * L, L)] = zero_i
        return _

    lax.fori_loop(0, 3 * 128 // L, zinit, 0)

    def zsrc(t):
        return z_hbm.at[pl.ds(t * ZBLK, ZBLK), pl.ds(w * 128, 128)]

    pltpu.async_copy(zsrc(0), zbuf.at[0], sems.at[0])

    def zblock(blk, _):
        cur = blk & 1
        pltpu.make_async_copy(zsrc(blk), zbuf.at[cur], sems.at[cur]).wait()

        @pl.when(blk + 1 < ZNBLK)
        def _prefetch():
            pltpu.async_copy(zsrc(blk + 1), zbuf.at[1 - cur], sems.at[1 - cur])

        rbase = blk * ZBLK

        def zgroup(g, _):
            t0 = zsv[pl.ds(0 * 128 + g * L, L)]
            t1 = zsv[pl.ds(1 * 128 + g * L, L)]
            t2 = zsv[pl.ds(2 * 128 + g * L, L)]
            i0 = zsi[pl.ds(0 * 128 + g * L, L)]
            i1 = zsi[pl.ds(1 * 128 + g * L, L)]
            i2 = zsi[pl.ds(2 * 128 + g * L, L)]

            v0 = zbuf[cur, 0, pl.ds(g * L, L)]

            def zstep(r, carry):
                t0, i0, t1, i1, t2, i2, v = carry
                rn = jnp.minimum(r + 1, ZBLK - 1)
                vnext = zbuf[cur, rn, pl.ds(g * L, L)]
                iv = jnp.full((L,), rbase + r, jnp.int32)
                out = _insert3_desc(t0, i0, t1, i1, t2, i2, v, iv)
                return out + (vnext,)

            t0, i0, t1, i1, t2, i2, _vlast = lax.fori_loop(
                0, ZBLK, zstep, (t0, i0, t1, i1, t2, i2, v0))
            zsv[pl.ds(0 * 128 + g * L, L)] = t0
            zsv[pl.ds(1 * 128 + g * L, L)] = t1
            zsv[pl.ds(2 * 128 + g * L, L)] = t2
            zsi[pl.ds(0 * 128 + g * L, L)] = i0
            zsi[pl.ds(1 * 128 + g * L, L)] = i1
            zsi[pl.ds(2 * 128 + g * L, L)] = i2
            return _

        lax.fori_loop(0, 8, zgroup, 0)
        return _

    lax.fori_loop(0, ZNBLK, zblock, 0)
    pltpu.sync_copy(zsv, zv_o.at[pl.ds(w * 384, 384)])
    pltpu.sync_copy(zsi, zi_o.at[pl.ds(w * 384, 384)])
    phase_z.__exit__(None, None, None)


@jax.jit
def _sc_topk(x, y2d, z):
    mesh = plsc.VectorSubcoreMesh(core_axis_name="c", subcore_axis_name="s")
    f = pl.kernel(
        _sc_body,
        mesh=mesh,
        out_type=[
            jax.ShapeDtypeStruct((NW * L,), jnp.float32),
            jax.ShapeDtypeStruct((NW * L,), jnp.int32),
            jax.ShapeDtypeStruct((4096 * 4,), jnp.float32),
            jax.ShapeDtypeStruct((4096 * 4,), jnp.int32),
            jax.ShapeDtypeStruct((NW * 384,), jnp.float32),
            jax.ShapeDtypeStruct((NW * 384,), jnp.int32),
        ],
        scratch_types=[
            pltpu.VMEM((2, XCHUNK), jnp.float32),
            pltpu.VMEM((L, 4096), jnp.float32),
            pltpu.VMEM((YNBLK, L), jnp.float32),
            pltpu.VMEM((2, ZBLK, 128), jnp.float32),
            pltpu.VMEM((384,), jnp.float32),
            pltpu.VMEM((384,), jnp.int32),
            pltpu.VMEM((4 * L,), jnp.float32),
            pltpu.VMEM((4 * L,), jnp.int32),
            pltpu.SemaphoreType.DMA((3,)),
        ],
        compiler_params=pltpu.CompilerParams(needs_layout_passes=False),
    )
    return f(x, y2d, z)


def kernel(x, y, z):
    y2d = y.reshape(4096, 4096)
    xv, xi, yv, yi, zv, zi = _sc_topk(x, y2d, z)
    x_values = xv.reshape(NW, L)[:, :4].reshape(64, 2)
    x_indices = xi.reshape(NW, L)[:, :4].reshape(64, 2).astype(jnp.int64)
    y_values = yv.reshape(32, 16, 8, 4)
    y_indices = yi.reshape(32, 16, 8, 4).astype(jnp.int64)
    z_values = zv.reshape(NW, 3, 128).transpose(1, 0, 2).reshape(3, 4096)
    z_indices = zi.reshape(NW, 3, 128).transpose(1, 0, 2).reshape(3, 4096).astype(jnp.int64)
    return (x_values, x_indices, y_values, y_indices, z_values, z_indices)
